# baseline (speedup 1.0000x reference)
import functools

import jax
import jax.numpy as jnp
from jax.experimental import pallas as pl
from jax.experimental.pallas import tpu as pltpu

N_LAYERS = 3
EPS = 1e-12
BS = 256
NSLOTS = 4
NT = 512


def _fused_kernel(adj_m_hbm, adj_a_hbm, emb_m_hbm, emb_a_hbm, out_ref,
                  adj_bf, et_a, et_b, emb_vmem, chunk_buf, sems, emb_sem,
                  *, n, n_strips, nt, n_nt):
    g = pl.program_id(0)

    @pl.when(g == 0)
    def _():
        pltpu.make_async_copy(emb_m_hbm, emb_vmem, emb_sem).start()

    @pl.when(g != 0)
    def _():
        pltpu.make_async_copy(emb_a_hbm, emb_vmem, emb_sem).start()

    def start_copy(i, slot):
        w = (i + 1) * BS

        @pl.when(g == 0)
        def _():
            pltpu.make_async_copy(
                adj_m_hbm.at[pl.ds(i * BS, BS), pl.ds(0, w)],
                chunk_buf.at[slot, :, pl.ds(0, w)], sems.at[slot]).start()

        @pl.when(g != 0)
        def _():
            pltpu.make_async_copy(
                adj_a_hbm.at[pl.ds(i * BS, BS), pl.ds(0, w)],
                chunk_buf.at[slot, :, pl.ds(0, w)], sems.at[slot]).start()

    def wait_copy(i, slot):
        w = (i + 1) * BS
        buf = chunk_buf.at[slot, :, pl.ds(0, w)]
        pltpu.make_async_copy(buf, buf, sems.at[slot]).wait()

    for s in range(min(NSLOTS - 1, n_strips)):
        i0 = n_strips - 1 - s
        start_copy(i0, i0 % NSLOTS)
    et_a[...] = jnp.zeros_like(et_a)
    pltpu.make_async_copy(emb_vmem, emb_vmem, emb_sem).wait()
    e0_bf = emb_vmem[...].astype(jnp.bfloat16)

    order = list(range(n_strips - 1, -1, -1))
    for k, i in enumerate(order):
        if k + NSLOTS - 1 < n_strips:
            nxt = order[k + NSLOTS - 1]
            start_copy(nxt, nxt % NSLOTS)
        wait_copy(i, i % NSLOTS)
        w = (i + 1) * BS
        rows = pl.ds(i * BS, BS)
        strip_bf = chunk_buf[i % NSLOTS, :, :w].astype(jnp.bfloat16)
        adj_bf[rows, pl.ds(0, w)] = strip_bf
        et_a[:, pl.ds(0, w)] += jax.lax.dot_general(
            e0_bf[i * BS:(i + 1) * BS, :], strip_bf, (((0,), (0,)), ((), ())),
            preferred_element_type=jnp.float32)
        if i > 0:
            t_bf = jnp.transpose(strip_bf[:, :i * BS])
            adj_bf[pl.ds(0, i * BS), rows] = t_bf
            et_a[:, rows] += jax.lax.dot_general(
                e0_bf[:i * BS, :], strip_bf[:, :i * BS],
                (((0,), (1,)), ((), ())),
                preferred_element_type=jnp.float32)

    def layer(e_in, e_out):
        e_bf = e_in[...].astype(jnp.bfloat16)
        for t in range(n_nt):
            cols = pl.ds(t * nt, nt)
            r = jnp.dot(e_bf, adj_bf[:, cols],
                        preferred_element_type=jnp.float32)
            if e_out is not None:
                e_out[:, cols] = r
            et_a[:, cols] += r

    layer(et_a, et_b)
    layer(et_b, None)

    for t in range(n_nt):
        rows = pl.ds(t * nt, nt)
        acc_t = jnp.transpose(et_a[:, rows])
        out_ref[0, rows, :] = acc_t + emb_vmem[rows, :]

    x = out_ref[0]
    sq = jnp.sum(x * x, axis=1, keepdims=True)
    inv = jax.lax.rsqrt(jnp.maximum(sq, EPS * EPS))
    out_ref[0] = x * inv


def kernel(adj_mashup, adj_api, mashup_emb, api_emb):
    n, d = mashup_emb.shape
    assert adj_mashup.shape == (n, n) and adj_api.shape == (n, n)
    assert n % BS == 0 and n % NT == 0

    body = functools.partial(_fused_kernel, n=n, n_strips=n // BS,
                             nt=NT, n_nt=n // NT)
    out = pl.pallas_call(
        body,
        out_shape=jax.ShapeDtypeStruct((2, n, d), jnp.float32),
        grid=(2,),
        in_specs=[
            pl.BlockSpec(memory_space=pl.ANY),
            pl.BlockSpec(memory_space=pl.ANY),
            pl.BlockSpec(memory_space=pl.ANY),
            pl.BlockSpec(memory_space=pl.ANY),
        ],
        out_specs=pl.BlockSpec((1, n, d), lambda g: (g, 0, 0)),
        scratch_shapes=[
            pltpu.VMEM((n, n), jnp.bfloat16),
            pltpu.VMEM((d, n), jnp.float32),
            pltpu.VMEM((d, n), jnp.float32),
            pltpu.VMEM((n, d), jnp.float32),
            pltpu.VMEM((NSLOTS, BS, n), jnp.float32),
            pltpu.SemaphoreType.DMA((NSLOTS,)),
            pltpu.SemaphoreType.DMA,
        ],
        compiler_params=pltpu.CompilerParams(
            dimension_semantics=("parallel",),
            vmem_limit_bytes=62 * 1024 * 1024,
        ),
    )(adj_mashup.astype(jnp.float32), adj_api.astype(jnp.float32),
      mashup_emb.astype(jnp.float32), api_emb.astype(jnp.float32))
    return out[0], out[1]


# speedup vs baseline: 1.2421x; 1.2421x over previous
import functools

import jax
import jax.numpy as jnp
from jax.experimental import pallas as pl
from jax.experimental.pallas import tpu as pltpu

N_LAYERS = 3
EPS = 1e-12
BS = 256
NSLOTS = 4
NT = 512


def _fused_kernel(adj_m_hbm, adj_a_hbm, emb_m_hbm, emb_a_hbm, out_ref,
                  adj_bf, et_a, et_b, emb_vmem, chunk_buf, sems, emb_sem,
                  *, n, n_strips, nt, n_nt):
    g = pl.program_id(0)

    @pl.when(g == 0)
    def _():
        pltpu.make_async_copy(emb_m_hbm, emb_vmem, emb_sem).start()

    @pl.when(g != 0)
    def _():
        pltpu.make_async_copy(emb_a_hbm, emb_vmem, emb_sem).start()

    def start_copy(i, slot):
        w = (i + 1) * BS

        @pl.when(g == 0)
        def _():
            pltpu.make_async_copy(
                adj_m_hbm.at[pl.ds(i * BS, BS), pl.ds(0, w)],
                chunk_buf.at[slot, :, pl.ds(0, w)], sems.at[slot]).start()

        @pl.when(g != 0)
        def _():
            pltpu.make_async_copy(
                adj_a_hbm.at[pl.ds(i * BS, BS), pl.ds(0, w)],
                chunk_buf.at[slot, :, pl.ds(0, w)], sems.at[slot]).start()

    def wait_copy(i, slot):
        w = (i + 1) * BS
        buf = chunk_buf.at[slot, :, pl.ds(0, w)]
        pltpu.make_async_copy(buf, buf, sems.at[slot]).wait()

    for s in range(min(NSLOTS - 1, n_strips)):
        start_copy(s, s)
    et_a[...] = jnp.zeros_like(et_a)
    pltpu.make_async_copy(emb_vmem, emb_vmem, emb_sem).wait()
    e0_bf = emb_vmem[...].astype(jnp.bfloat16)

    for i in range(n_strips):
        if i + NSLOTS - 1 < n_strips:
            start_copy(i + NSLOTS - 1, (i + NSLOTS - 1) % NSLOTS)
        wait_copy(i, i % NSLOTS)
        w = (i + 1) * BS
        rows = pl.ds(i * BS, BS)
        strip_bf = chunk_buf[i % NSLOTS, :, :w].astype(jnp.bfloat16)
        adj_bf[rows, pl.ds(0, w)] = strip_bf
        et_a[:, pl.ds(0, w)] += jax.lax.dot_general(
            e0_bf[i * BS:(i + 1) * BS, :], strip_bf, (((0,), (0,)), ((), ())),
            preferred_element_type=jnp.float32)
        if i > 0:
            t_bf = jnp.transpose(strip_bf[:, :i * BS])
            adj_bf[pl.ds(0, i * BS), rows] = t_bf
            et_a[:, rows] += jax.lax.dot_general(
                e0_bf[:i * BS, :], t_bf, (((0,), (0,)), ((), ())),
                preferred_element_type=jnp.float32)

    def layer(e_in, e_out):
        e_bf = e_in[...].astype(jnp.bfloat16)
        for t in range(n_nt):
            cols = pl.ds(t * nt, nt)
            r = jnp.dot(e_bf, adj_bf[:, cols],
                        preferred_element_type=jnp.float32)
            if e_out is not None:
                e_out[:, cols] = r
            et_a[:, cols] += r

    layer(et_a, et_b)
    layer(et_b, None)

    for t in range(n_nt):
        rows = pl.ds(t * nt, nt)
        acc_t = jnp.transpose(et_a[:, rows])
        out_ref[0, rows, :] = acc_t + emb_vmem[rows, :]

    x = out_ref[0]
    sq = jnp.sum(x * x, axis=1, keepdims=True)
    inv = jax.lax.rsqrt(jnp.maximum(sq, EPS * EPS))
    out_ref[0] = x * inv


def kernel(adj_mashup, adj_api, mashup_emb, api_emb):
    n, d = mashup_emb.shape
    assert adj_mashup.shape == (n, n) and adj_api.shape == (n, n)
    assert n % BS == 0 and n % NT == 0

    body = functools.partial(_fused_kernel, n=n, n_strips=n // BS,
                             nt=NT, n_nt=n // NT)
    out = pl.pallas_call(
        body,
        out_shape=jax.ShapeDtypeStruct((2, n, d), jnp.float32),
        grid=(2,),
        in_specs=[
            pl.BlockSpec(memory_space=pl.ANY),
            pl.BlockSpec(memory_space=pl.ANY),
            pl.BlockSpec(memory_space=pl.ANY),
            pl.BlockSpec(memory_space=pl.ANY),
        ],
        out_specs=pl.BlockSpec((1, n, d), lambda g: (g, 0, 0)),
        scratch_shapes=[
            pltpu.VMEM((n, n), jnp.bfloat16),
            pltpu.VMEM((d, n), jnp.float32),
            pltpu.VMEM((d, n), jnp.float32),
            pltpu.VMEM((n, d), jnp.float32),
            pltpu.VMEM((NSLOTS, BS, n), jnp.float32),
            pltpu.SemaphoreType.DMA((NSLOTS,)),
            pltpu.SemaphoreType.DMA,
        ],
        compiler_params=pltpu.CompilerParams(
            dimension_semantics=("parallel",),
            vmem_limit_bytes=62 * 1024 * 1024,
        ),
    )(adj_mashup.astype(jnp.float32), adj_api.astype(jnp.float32),
      mashup_emb.astype(jnp.float32), api_emb.astype(jnp.float32))
    return out[0], out[1]
